# speedup vs baseline: 3.5956x; 3.5956x over previous
import jax
import jax.numpy as jnp
from jax import lax
from jax.experimental import pallas as pl
from jax.experimental.pallas import tpu as pltpu
from jax.experimental.pallas import tpu_sc as plsc

B = 1024
V = 100000
VP = 100352
NT = 16
CHUNKS = 49
CW = 128
TILE_N = CHUNKS * CW
BLK = 1024
GRID = VP // BLK


def _colstats_body(x_ref, s1_ref, s2_ref):
    x = x_ref[...]
    s1 = jnp.sum(x, axis=0, keepdims=True)
    s2 = jnp.sum(x * x, axis=0, keepdims=True)
    col = pl.program_id(0) * BLK + lax.broadcasted_iota(jnp.int32, (1, BLK), 1)
    valid = col < V
    s1 = jnp.where(valid, s1, 0.0)
    s2 = jnp.where(valid, s2, 0.0)
    s1_ref[...] = jnp.broadcast_to(s1, (8, BLK))
    s2_ref[...] = jnp.broadcast_to(s2, (8, BLK))


def _colstats(x):
    return pl.pallas_call(
        _colstats_body,
        grid=(GRID,),
        in_specs=[pl.BlockSpec((B, BLK), lambda i: (0, i))],
        out_specs=[
            pl.BlockSpec((8, BLK), lambda i: (0, i)),
            pl.BlockSpec((8, BLK), lambda i: (0, i)),
        ],
        out_shape=[
            jax.ShapeDtypeStruct((8, VP), jnp.float32),
            jax.ShapeDtypeStruct((8, VP), jnp.float32),
        ],
    )(x)


def _sc_body(s1_hbm, s2_hbm, tgt_hbm, z_hbm, out_hbm,
             idx_v, g1_v, g2_v, m_v, ones_v, acc_v, mask_sh, sem):
    wid = lax.axis_index("s")
    base = wid * TILE_N

    pltpu.sync_copy(tgt_hbm.at[wid], idx_v)
    pltpu.sync_copy(z_hbm.at[wid], mask_sh.at[pl.ds(base, TILE_N)])
    for i in range(CW // 16):
        ones_v[pl.ds(i * 16, 16)] = jnp.ones((16,), jnp.float32)
    plsc.subcore_barrier()

    def scat(k, c):
        pltpu.sync_copy(ones_v, mask_sh.at[idx_v.at[k]])
        return c
    lax.fori_loop(0, CHUNKS, scat, 0)
    plsc.subcore_barrier()

    def gat(k, acc):
        pltpu.async_copy(s1_hbm.at[idx_v.at[k]], g1_v, sem).wait()
        pltpu.async_copy(s2_hbm.at[idx_v.at[k]], g2_v, sem).wait()
        pltpu.sync_copy(mask_sh.at[pl.ds(base + k * CW, CW)], m_v)

        def inner(i, a):
            g1 = g1_v[pl.ds(i * 16, 16)]
            g2 = g2_v[pl.ds(i * 16, 16)]
            l = 2.0 * m_v[pl.ds(i * 16, 16)] - 1.0
            return a + (g2 - 2.0 * l * g1)
        return lax.fori_loop(0, CW // 16, inner, acc)

    acc = lax.fori_loop(0, CHUNKS, gat, jnp.zeros((16,), jnp.float32))
    acc_v[...] = acc
    pltpu.sync_copy(acc_v, out_hbm.at[wid])


_sc_loss = pl.kernel(
    _sc_body,
    out_type=jax.ShapeDtypeStruct((NT, 16), jnp.float32),
    mesh=plsc.VectorSubcoreMesh(
        core_axis_name="c", subcore_axis_name="s", num_cores=1),
    scratch_types=[
        pltpu.VMEM((CHUNKS, CW), jnp.int32),
        pltpu.VMEM((CW,), jnp.float32),
        pltpu.VMEM((CW,), jnp.float32),
        pltpu.VMEM((CW,), jnp.float32),
        pltpu.VMEM((CW,), jnp.float32),
        pltpu.VMEM((16,), jnp.float32),
        pltpu.VMEM_SHARED((VP,), jnp.float32),
        pltpu.SemaphoreType.DMA,
    ],
)


@jax.jit
def kernel(inputs, targets):
    s1_8, s2_8 = _colstats(inputs)
    s1 = s1_8[0]
    s2 = s2_8[0]
    tgt = jnp.concatenate(
        [targets.astype(jnp.int32),
         jnp.full((VP - V,), V, jnp.int32)]).reshape(NT, CHUNKS, CW)
    zeros = jnp.zeros((NT, TILE_N), jnp.float32)
    partials = _sc_loss(s1, s2, tgt, zeros)
    return jnp.sum(partials) / (B * V) + 1.0


# speedup vs baseline: 3.7286x; 1.0370x over previous
import jax
import jax.numpy as jnp
from jax import lax
from jax.experimental import pallas as pl
from jax.experimental.pallas import tpu as pltpu
from jax.experimental.pallas import tpu_sc as plsc

B = 1024
V = 100000
VP = 100352
NT = 16
CHUNKS = 49
CW = 128
TILE_N = CHUNKS * CW
BLK = 2048
GRID = VP // BLK


def _colstats_body(x_ref, s1_ref, s2_ref):
    x = x_ref[...]
    s1 = jnp.sum(x, axis=0, keepdims=True)
    s2 = jnp.sum(x * x, axis=0, keepdims=True)
    col = pl.program_id(0) * BLK + lax.broadcasted_iota(jnp.int32, (1, BLK), 1)
    valid = col < V
    s1 = jnp.where(valid, s1, 0.0)
    s2 = jnp.where(valid, s2, 0.0)
    s1_ref[...] = jnp.broadcast_to(s1, (8, BLK))
    s2_ref[...] = jnp.broadcast_to(s2, (8, BLK))


def _colstats(x):
    return pl.pallas_call(
        _colstats_body,
        grid=(GRID,),
        in_specs=[pl.BlockSpec((B, BLK), lambda i: (0, i))],
        out_specs=[
            pl.BlockSpec((8, BLK), lambda i: (0, i)),
            pl.BlockSpec((8, BLK), lambda i: (0, i)),
        ],
        out_shape=[
            jax.ShapeDtypeStruct((8, VP), jnp.float32),
            jax.ShapeDtypeStruct((8, VP), jnp.float32),
        ],
    )(x)


def _sc_body(s1_hbm, s2_hbm, tgt_hbm, z_hbm, out_hbm,
             idx_v, g1_v, g2_v, m_v, ones_v, acc_v, mask_sh, sem):
    wid = lax.axis_index("s")
    base = wid * TILE_N

    pltpu.sync_copy(tgt_hbm.at[wid], idx_v)
    pltpu.sync_copy(z_hbm.at[wid], mask_sh.at[pl.ds(base, TILE_N)])
    for i in range(CW // 16):
        ones_v[pl.ds(i * 16, 16)] = jnp.ones((16,), jnp.float32)
    plsc.subcore_barrier()

    def scat(k, c):
        pltpu.sync_copy(ones_v, mask_sh.at[idx_v.at[k]])
        return c
    lax.fori_loop(0, CHUNKS, scat, 0)
    plsc.subcore_barrier()

    def gat(k, acc):
        pltpu.async_copy(s1_hbm.at[idx_v.at[k]], g1_v, sem).wait()
        pltpu.async_copy(s2_hbm.at[idx_v.at[k]], g2_v, sem).wait()
        pltpu.sync_copy(mask_sh.at[pl.ds(base + k * CW, CW)], m_v)

        def inner(i, a):
            g1 = g1_v[pl.ds(i * 16, 16)]
            g2 = g2_v[pl.ds(i * 16, 16)]
            l = 2.0 * m_v[pl.ds(i * 16, 16)] - 1.0
            return a + (g2 - 2.0 * l * g1)
        return lax.fori_loop(0, CW // 16, inner, acc)

    acc = lax.fori_loop(0, CHUNKS, gat, jnp.zeros((16,), jnp.float32))
    acc_v[...] = acc
    pltpu.sync_copy(acc_v, out_hbm.at[wid])


_sc_loss = pl.kernel(
    _sc_body,
    out_type=jax.ShapeDtypeStruct((NT, 16), jnp.float32),
    mesh=plsc.VectorSubcoreMesh(
        core_axis_name="c", subcore_axis_name="s", num_cores=1),
    scratch_types=[
        pltpu.VMEM((CHUNKS, CW), jnp.int32),
        pltpu.VMEM((CW,), jnp.float32),
        pltpu.VMEM((CW,), jnp.float32),
        pltpu.VMEM((CW,), jnp.float32),
        pltpu.VMEM((CW,), jnp.float32),
        pltpu.VMEM((16,), jnp.float32),
        pltpu.VMEM_SHARED((VP,), jnp.float32),
        pltpu.SemaphoreType.DMA,
    ],
)


@jax.jit
def kernel(inputs, targets):
    s1_8, s2_8 = _colstats(inputs)
    s1 = s1_8[0]
    s2 = s2_8[0]
    tgt = jnp.concatenate(
        [targets.astype(jnp.int32),
         jnp.full((VP - V,), V, jnp.int32)]).reshape(NT, CHUNKS, CW)
    zeros = jnp.zeros((NT, TILE_N), jnp.float32)
    partials = _sc_loss(s1, s2, tgt, zeros)
    return jnp.sum(partials) / (B * V) + 1.0
